# baseline (speedup 1.0000x reference)
import functools

import jax
import jax.numpy as jnp
from jax import lax
from jax.experimental import pallas as pl
from jax.experimental.pallas import tpu as pltpu
from jax.experimental.pallas import tpu_sc as plsc

N_NODES = 10000
N_EDGES = 320000
D = 128

NP = 10240
EP = 327680
N_TILES = 32
EPT = EP // N_TILES
CHUNK = 128
NCH = EPT // CHUNK
HCH = NCH // 2
RPS = NP // 16
NBUF = 2
TB = 512

_mesh = plsc.VectorSubcoreMesh(core_axis_name="c", subcore_axis_name="s")
_sc_params = pltpu.CompilerParams(needs_layout_passes=False)


@functools.partial(
    pl.kernel,
    out_type=(
        jax.ShapeDtypeStruct((N_TILES, NP), jnp.float32),
        jax.ShapeDtypeStruct((N_TILES, NP), jnp.float32),
    ),
    mesh=_mesh,
    scratch_types=[
        pltpu.VMEM((EPT,), jnp.int32),
        pltpu.VMEM((EPT,), jnp.int32),
        pltpu.VMEM((NP,), jnp.float32),
        pltpu.VMEM((NP,), jnp.float32),
    ],
    compiler_params=_sc_params,
)
def _deg_kernel(src_hbm, dst_hbm, z1_hbm, outs_hbm, outd_hbm, src_v, dst_v,
                hs_v, hd_v):
    w = lax.axis_index("s") * 2 + lax.axis_index("c")
    pltpu.sync_copy(z1_hbm, hs_v)
    pltpu.sync_copy(z1_hbm, hd_v)
    pltpu.sync_copy(src_hbm.at[w], src_v)
    pltpu.sync_copy(dst_hbm.at[w], dst_v)
    ones = jnp.ones((16,), jnp.float32)

    def body(g, carry):
        base = g * 64
        for u in range(4):
            idx16 = src_v[pl.ds(base + u * 16, 16)]
            plsc.addupdate_scatter(hs_v, [idx16], ones)
            idx16 = dst_v[pl.ds(base + u * 16, 16)]
            plsc.addupdate_scatter(hd_v, [idx16], ones)
        return carry

    lax.fori_loop(0, EPT // 64, body, 0)
    pltpu.sync_copy(hs_v, outs_hbm.at[w])
    pltpu.sync_copy(hd_v, outd_hbm.at[w])


@functools.partial(
    pl.kernel,
    out_type=jax.ShapeDtypeStruct((2, NP, D), jnp.float32),
    mesh=_mesh,
    scratch_types=[
        pltpu.VMEM((HCH, CHUNK), jnp.int32),
        pltpu.VMEM((HCH, CHUNK), jnp.int32),
        pltpu.VMEM((NBUF, CHUNK, D), jnp.float32),
        pltpu.VMEM_SHARED((NP, D), jnp.float32),
        pltpu.SemaphoreType.DMA,
        pltpu.SemaphoreType.DMA,
    ],
    compiler_params=_sc_params,
)
def _prop_kernel(t_hbm, src_hbm, dst_hbm, z2_hbm, out_hbm, si_v, di_v, rows_v,
                 acc_sh, sem0, sem1):
    sems = [sem0, sem1]
    c = lax.axis_index("c")
    s = lax.axis_index("s")
    w = s * 2 + c
    pltpu.sync_copy(z2_hbm, acc_sh.at[pl.ds(s * RPS, RPS)])

    def gstart(b, j):
        pltpu.async_copy(t_hbm.at[si_v.at[j]], rows_v.at[b], sems[b])

    def gwait(b):
        pltpu.make_async_copy(t_hbm.at[si_v.at[0]], rows_v.at[b],
                              sems[b]).wait()

    plsc.subcore_barrier()

    for hh in range(2):
        pltpu.sync_copy(src_hbm.at[w, pl.ds(hh * HCH, HCH)], si_v)
        pltpu.sync_copy(dst_hbm.at[w, pl.ds(hh * HCH, HCH)], di_v)
        for b in range(NBUF):
            gstart(b, b)

        def body(k, carry):
            for i in range(NBUF):
                t = k * NBUF + i
                gwait(i)
                pltpu.sync_copy(rows_v.at[i], acc_sh.at[di_v.at[t]], add=True)
                gstart(i, t + NBUF)
            return carry

        lax.fori_loop(0, (HCH - NBUF) // NBUF, body, 0)
        for i in range(NBUF):
            t = HCH - NBUF + i
            gwait(i)
            pltpu.sync_copy(rows_v.at[i], acc_sh.at[di_v.at[t]], add=True)

    plsc.subcore_barrier()
    pltpu.sync_copy(acc_sh.at[pl.ds(s * RPS, RPS)],
                    out_hbm.at[c, pl.ds(s * RPS, RPS)])


def _norm_prescale_body(ps_ref, pd_ref, x_ref, t0_ref, ns_ref, nd_ref):
    degs = jnp.sum(ps_ref[...], axis=0)
    degd = jnp.sum(pd_ref[...], axis=0)
    nsv = lax.rsqrt(jnp.maximum(degs, 1.0))
    ndv = lax.rsqrt(jnp.maximum(degd, 1.0))
    t0_ref[...] = x_ref[...] * nsv[:, None]
    ns_ref[...] = nsv[:, None]
    nd_ref[...] = ndv[:, None]


_norm_prescale = pl.pallas_call(
    _norm_prescale_body,
    grid=(NP // TB,),
    in_specs=[
        pl.BlockSpec((N_TILES, TB), lambda i: (0, i)),
        pl.BlockSpec((N_TILES, TB), lambda i: (0, i)),
        pl.BlockSpec((TB, D), lambda i: (i, 0)),
    ],
    out_specs=[
        pl.BlockSpec((TB, D), lambda i: (i, 0)),
        pl.BlockSpec((TB, 1), lambda i: (i, 0)),
        pl.BlockSpec((TB, 1), lambda i: (i, 0)),
    ],
    out_shape=[
        jax.ShapeDtypeStruct((NP, D), jnp.float32),
        jax.ShapeDtypeStruct((NP, 1), jnp.float32),
        jax.ShapeDtypeStruct((NP, 1), jnp.float32),
    ],
)


def _mid_body(acc_ref, nd_ref, ns_ref, w_ref, b_ref, t1_ref):
    a = (acc_ref[0] + acc_ref[1]) * nd_ref[...]
    h = jnp.dot(a, w_ref[...], preferred_element_type=jnp.float32) + b_ref[...]
    t1_ref[...] = h * ns_ref[...]


_mid_layer = pl.pallas_call(
    _mid_body,
    grid=(NP // TB,),
    in_specs=[
        pl.BlockSpec((2, TB, D), lambda i: (0, i, 0)),
        pl.BlockSpec((TB, 1), lambda i: (i, 0)),
        pl.BlockSpec((TB, 1), lambda i: (i, 0)),
        pl.BlockSpec((D, D), lambda i: (0, 0)),
        pl.BlockSpec((1, D), lambda i: (0, 0)),
    ],
    out_specs=pl.BlockSpec((TB, D), lambda i: (i, 0)),
    out_shape=jax.ShapeDtypeStruct((NP, D), jnp.float32),
)


def _out_body(acc_ref, nd_ref, w_ref, b_ref, o_ref):
    a = (acc_ref[0] + acc_ref[1]) * nd_ref[...]
    o_ref[...] = jnp.dot(a, w_ref[...], preferred_element_type=jnp.float32) + b_ref[...]


_out_layer = pl.pallas_call(
    _out_body,
    grid=(NP // TB,),
    in_specs=[
        pl.BlockSpec((2, TB, D), lambda i: (0, i, 0)),
        pl.BlockSpec((TB, 1), lambda i: (i, 0)),
        pl.BlockSpec((D, D), lambda i: (0, 0)),
        pl.BlockSpec((1, D), lambda i: (0, 0)),
    ],
    out_specs=pl.BlockSpec((TB, D), lambda i: (i, 0)),
    out_shape=jax.ShapeDtypeStruct((NP, D), jnp.float32),
)


def kernel(x, edge_index, W1, b1, W2, b2):
    src = edge_index[0].astype(jnp.int32)
    dst = edge_index[1].astype(jnp.int32)
    padi = jnp.full((EP - N_EDGES,), NP - 1, jnp.int32)
    src_p = jnp.concatenate([src, padi])
    dst_p = jnp.concatenate([dst, padi])
    src2 = src_p.reshape(N_TILES, EPT)
    dst2 = dst_p.reshape(N_TILES, EPT)
    src3 = src_p.reshape(N_TILES, NCH, CHUNK)
    dst3 = dst_p.reshape(N_TILES, NCH, CHUNK)
    z1 = jnp.zeros((NP,), jnp.float32)
    z2 = jnp.zeros((RPS, D), jnp.float32)
    x_p = jnp.pad(x, ((0, NP - N_NODES), (0, 0)))

    hs, hd = _deg_kernel(src2, dst2, z1)
    t0, ns, nd = _norm_prescale(hs, hd, x_p)
    acc1 = _prop_kernel(t0, src3, dst3, z2)
    t1 = _mid_layer(acc1, nd, ns, W1, b1.reshape(1, D))
    acc2 = _prop_kernel(t1, src3, dst3, z2)
    out = _out_layer(acc2, nd, W2, b2.reshape(1, D))
    return out[:N_NODES]


# speedup vs baseline: 3.1935x; 3.1935x over previous
import functools

import jax
import jax.numpy as jnp
from jax import lax
from jax.experimental import pallas as pl
from jax.experimental.pallas import tpu as pltpu
from jax.experimental.pallas import tpu_sc as plsc

N_NODES = 10000
N_EDGES = 320000
D = 128

NP = 10240
EP = 327680
N_TILES = 32
EPT = EP // N_TILES
CHUNK = 128
NCH = EPT // CHUNK
HCH = NCH // 2
RPS = NP // 16
NBUF = 2
TB = 512

_mesh = plsc.VectorSubcoreMesh(core_axis_name="c", subcore_axis_name="s")
_sc_params = pltpu.CompilerParams(needs_layout_passes=False)


@functools.partial(
    pl.kernel,
    out_type=(
        jax.ShapeDtypeStruct((N_TILES, NP), jnp.float32),
        jax.ShapeDtypeStruct((N_TILES, NP), jnp.float32),
    ),
    mesh=_mesh,
    scratch_types=[
        pltpu.VMEM((EPT,), jnp.int32),
        pltpu.VMEM((EPT,), jnp.int32),
        pltpu.VMEM((NP,), jnp.float32),
        pltpu.VMEM((NP,), jnp.float32),
    ],
    compiler_params=_sc_params,
)
def _deg_kernel(src_hbm, dst_hbm, z1_hbm, outs_hbm, outd_hbm, src_v, dst_v,
                hs_v, hd_v):
    w = lax.axis_index("s") * 2 + lax.axis_index("c")
    pltpu.sync_copy(z1_hbm, hs_v)
    pltpu.sync_copy(z1_hbm, hd_v)
    pltpu.sync_copy(src_hbm.at[w], src_v)
    pltpu.sync_copy(dst_hbm.at[w], dst_v)
    ones = jnp.ones((16,), jnp.float32)

    def body(g, carry):
        base = g * 64
        for u in range(4):
            idx16 = src_v[pl.ds(base + u * 16, 16)]
            plsc.addupdate_scatter(hs_v, [idx16], ones)
            idx16 = dst_v[pl.ds(base + u * 16, 16)]
            plsc.addupdate_scatter(hd_v, [idx16], ones)
        return carry

    lax.fori_loop(0, EPT // 64, body, 0)
    pltpu.sync_copy(hs_v, outs_hbm.at[w])
    pltpu.sync_copy(hd_v, outd_hbm.at[w])


@functools.partial(
    pl.kernel,
    out_type=jax.ShapeDtypeStruct((2, NP, D), jnp.float32),
    mesh=_mesh,
    scratch_types=[
        pltpu.VMEM((HCH, CHUNK), jnp.int32),
        pltpu.VMEM((HCH, CHUNK), jnp.int32),
        pltpu.VMEM((NBUF, CHUNK, D), jnp.float32),
        pltpu.VMEM_SHARED((NP, D), jnp.float32),
        pltpu.SemaphoreType.DMA,
        pltpu.SemaphoreType.DMA,
    ],
    compiler_params=_sc_params,
)
def _prop_kernel(t_hbm, src_hbm, dst_hbm, z2_hbm, out_hbm, si_v, di_v, rows_v,
                 acc_sh, sem0, sem1):
    sems = [sem0, sem1]
    c = lax.axis_index("c")
    s = lax.axis_index("s")
    w = s * 2 + c
    pltpu.sync_copy(z2_hbm, acc_sh.at[pl.ds(s * RPS, RPS)])

    def gstart(b, j):
        pltpu.async_copy(t_hbm.at[si_v.at[j]], rows_v.at[b], sems[b])

    def gwait(b):
        pltpu.make_async_copy(t_hbm.at[si_v.at[0]], rows_v.at[b],
                              sems[b]).wait()

    plsc.subcore_barrier()

    for hh in range(2):
        pltpu.sync_copy(src_hbm.at[w, pl.ds(hh * HCH, HCH)], si_v)
        pltpu.sync_copy(dst_hbm.at[w, pl.ds(hh * HCH, HCH)], di_v)
        for b in range(NBUF):
            gstart(b, b)

        def body(k, carry):
            for i in range(NBUF):
                t = k * NBUF + i
                gwait(i)
                pltpu.sync_copy(rows_v.at[i], acc_sh.at[di_v.at[t]], add=True)
                gstart(i, t + NBUF)
            return carry

        lax.fori_loop(0, (HCH - NBUF) // NBUF, body, 0)
        for i in range(NBUF):
            t = HCH - NBUF + i
            gwait(i)
            pltpu.sync_copy(rows_v.at[i], acc_sh.at[di_v.at[t]], add=True)

    plsc.subcore_barrier()
    pltpu.sync_copy(acc_sh.at[pl.ds(s * RPS, RPS)],
                    out_hbm.at[c, pl.ds(s * RPS, RPS)])


def _norm_prescale_body(ps_ref, pd_ref, x_ref, t0_ref, ns_ref, nd_ref):
    degs = jnp.sum(ps_ref[...], axis=0)
    degd = jnp.sum(pd_ref[...], axis=0)
    nsv = lax.rsqrt(jnp.maximum(degs, 1.0))
    ndv = lax.rsqrt(jnp.maximum(degd, 1.0))
    t0_ref[...] = x_ref[...] * nsv[:, None]
    ns_ref[...] = nsv[:, None]
    nd_ref[...] = ndv[:, None]


_norm_prescale = pl.pallas_call(
    _norm_prescale_body,
    grid=(NP // TB,),
    in_specs=[
        pl.BlockSpec((N_TILES, TB), lambda i: (0, i)),
        pl.BlockSpec((N_TILES, TB), lambda i: (0, i)),
        pl.BlockSpec((TB, D), lambda i: (i, 0)),
    ],
    out_specs=[
        pl.BlockSpec((TB, D), lambda i: (i, 0)),
        pl.BlockSpec((TB, 1), lambda i: (i, 0)),
        pl.BlockSpec((TB, 1), lambda i: (i, 0)),
    ],
    out_shape=[
        jax.ShapeDtypeStruct((NP, D), jnp.float32),
        jax.ShapeDtypeStruct((NP, 1), jnp.float32),
        jax.ShapeDtypeStruct((NP, 1), jnp.float32),
    ],
)


def _mid_body(acc_ref, nd_ref, ns_ref, w_ref, b_ref, t1_ref):
    a = (acc_ref[0] + acc_ref[1]) * nd_ref[...]
    h = jnp.dot(a, w_ref[...], preferred_element_type=jnp.float32) + b_ref[...]
    t1_ref[...] = h * ns_ref[...]


_mid_layer = pl.pallas_call(
    _mid_body,
    grid=(NP // TB,),
    in_specs=[
        pl.BlockSpec((2, TB, D), lambda i: (0, i, 0)),
        pl.BlockSpec((TB, 1), lambda i: (i, 0)),
        pl.BlockSpec((TB, 1), lambda i: (i, 0)),
        pl.BlockSpec((D, D), lambda i: (0, 0)),
        pl.BlockSpec((1, D), lambda i: (0, 0)),
    ],
    out_specs=pl.BlockSpec((TB, D), lambda i: (i, 0)),
    out_shape=jax.ShapeDtypeStruct((NP, D), jnp.float32),
)


def _out_body(acc_ref, nd_ref, w_ref, b_ref, o_ref):
    a = (acc_ref[0] + acc_ref[1]) * nd_ref[...]
    o_ref[...] = jnp.dot(a, w_ref[...], preferred_element_type=jnp.float32) + b_ref[...]


_out_layer = pl.pallas_call(
    _out_body,
    grid=(NP // TB,),
    in_specs=[
        pl.BlockSpec((2, TB, D), lambda i: (0, i, 0)),
        pl.BlockSpec((TB, 1), lambda i: (i, 0)),
        pl.BlockSpec((D, D), lambda i: (0, 0)),
        pl.BlockSpec((1, D), lambda i: (0, 0)),
    ],
    out_specs=pl.BlockSpec((TB, D), lambda i: (i, 0)),
    out_shape=jax.ShapeDtypeStruct((NP, D), jnp.float32),
)


def kernel(x, edge_index, W1, b1, W2, b2):
    src = edge_index[0].astype(jnp.int32)
    dst = edge_index[1].astype(jnp.int32)
    padi = N_NODES + jnp.arange(EP - N_EDGES, dtype=jnp.int32) % (NP - N_NODES)
    src_p = jnp.concatenate([src, padi])
    dst_p = jnp.concatenate([dst, padi])
    src2 = src_p.reshape(N_TILES, EPT)
    dst2 = dst_p.reshape(N_TILES, EPT)
    src3 = src_p.reshape(N_TILES, NCH, CHUNK)
    dst3 = dst_p.reshape(N_TILES, NCH, CHUNK)
    z1 = jnp.zeros((NP,), jnp.float32)
    z2 = jnp.zeros((RPS, D), jnp.float32)
    x_p = jnp.pad(x, ((0, NP - N_NODES), (0, 0)))

    hs, hd = _deg_kernel(src2, dst2, z1)
    t0, ns, nd = _norm_prescale(hs, hd, x_p)
    acc1 = _prop_kernel(t0, src3, dst3, z2)
    t1 = _mid_layer(acc1, nd, ns, W1, b1.reshape(1, D))
    acc2 = _prop_kernel(t1, src3, dst3, z2)
    out = _out_layer(acc2, nd, W2, b2.reshape(1, D))
    return out[:N_NODES]


# speedup vs baseline: 3.2130x; 1.0061x over previous
import functools

import jax
import jax.numpy as jnp
from jax import lax
from jax.experimental import pallas as pl
from jax.experimental.pallas import tpu as pltpu
from jax.experimental.pallas import tpu_sc as plsc

N_NODES = 10000
N_EDGES = 320000
D = 128

NP = 10240
EP = 327680
N_TILES = 32
EPT = EP // N_TILES
CHUNK = 128
NCH = EPT // CHUNK
HCH = NCH // 2
RPS = NP // 16
NBUF = 2
TB = 512

_mesh = plsc.VectorSubcoreMesh(core_axis_name="c", subcore_axis_name="s")
_sc_params = pltpu.CompilerParams(needs_layout_passes=False)


@functools.partial(
    pl.kernel,
    out_type=(
        jax.ShapeDtypeStruct((N_TILES, NP), jnp.float32),
        jax.ShapeDtypeStruct((N_TILES, NP), jnp.float32),
    ),
    mesh=_mesh,
    scratch_types=[
        pltpu.VMEM((NCH, CHUNK), jnp.int32),
        pltpu.VMEM((NCH, CHUNK), jnp.int32),
        pltpu.VMEM((NP,), jnp.float32),
        pltpu.VMEM((NP,), jnp.float32),
    ],
    compiler_params=_sc_params,
)
def _deg_kernel(src_hbm, dst_hbm, z1_hbm, outs_hbm, outd_hbm, src_v, dst_v,
                hs_v, hd_v):
    w = lax.axis_index("s") * 2 + lax.axis_index("c")
    pltpu.sync_copy(z1_hbm, hs_v)
    pltpu.sync_copy(z1_hbm, hd_v)
    pltpu.sync_copy(src_hbm.at[w], src_v)
    pltpu.sync_copy(dst_hbm.at[w], dst_v)
    ones = jnp.ones((16,), jnp.float32)

    def body(j, carry):
        for u in range(CHUNK // 16):
            idx16 = src_v[j, pl.ds(u * 16, 16)]
            plsc.addupdate_scatter(hs_v, [idx16], ones)
            idx16 = dst_v[j, pl.ds(u * 16, 16)]
            plsc.addupdate_scatter(hd_v, [idx16], ones)
        return carry

    lax.fori_loop(0, NCH, body, 0)
    pltpu.sync_copy(hs_v, outs_hbm.at[w])
    pltpu.sync_copy(hd_v, outd_hbm.at[w])


@functools.partial(
    pl.kernel,
    out_type=jax.ShapeDtypeStruct((2, NP, D), jnp.float32),
    mesh=_mesh,
    scratch_types=[
        pltpu.VMEM((HCH, CHUNK), jnp.int32),
        pltpu.VMEM((HCH, CHUNK), jnp.int32),
        pltpu.VMEM((NBUF, CHUNK, D), jnp.float32),
        pltpu.VMEM_SHARED((NP, D), jnp.float32),
        pltpu.SemaphoreType.DMA,
        pltpu.SemaphoreType.DMA,
    ],
    compiler_params=_sc_params,
)
def _prop_kernel(t_hbm, src_hbm, dst_hbm, z2_hbm, out_hbm, si_v, di_v, rows_v,
                 acc_sh, sem0, sem1):
    sems = [sem0, sem1]
    c = lax.axis_index("c")
    s = lax.axis_index("s")
    w = s * 2 + c
    pltpu.sync_copy(z2_hbm, acc_sh.at[pl.ds(s * RPS, RPS)])

    def gstart(b, j):
        pltpu.async_copy(t_hbm.at[si_v.at[j]], rows_v.at[b], sems[b])

    def gwait(b):
        pltpu.make_async_copy(t_hbm.at[si_v.at[0]], rows_v.at[b],
                              sems[b]).wait()

    plsc.subcore_barrier()

    for hh in range(2):
        pltpu.sync_copy(src_hbm.at[w, pl.ds(hh * HCH, HCH)], si_v)
        pltpu.sync_copy(dst_hbm.at[w, pl.ds(hh * HCH, HCH)], di_v)
        for b in range(NBUF):
            gstart(b, b)

        def body(k, carry):
            for i in range(NBUF):
                t = k * NBUF + i
                gwait(i)
                pltpu.sync_copy(rows_v.at[i], acc_sh.at[di_v.at[t]], add=True)
                gstart(i, t + NBUF)
            return carry

        lax.fori_loop(0, (HCH - NBUF) // NBUF, body, 0)
        for i in range(NBUF):
            t = HCH - NBUF + i
            gwait(i)
            pltpu.sync_copy(rows_v.at[i], acc_sh.at[di_v.at[t]], add=True)

    plsc.subcore_barrier()
    pltpu.sync_copy(acc_sh.at[pl.ds(s * RPS, RPS)],
                    out_hbm.at[c, pl.ds(s * RPS, RPS)])


def _norm_prescale_body(ps_ref, pd_ref, x_ref, t0_ref, ns_ref, nd_ref):
    degs = jnp.sum(ps_ref[...], axis=0)
    degd = jnp.sum(pd_ref[...], axis=0)
    nsv = lax.rsqrt(jnp.maximum(degs, 1.0))
    ndv = lax.rsqrt(jnp.maximum(degd, 1.0))
    t0_ref[...] = x_ref[...] * nsv[:, None]
    ns_ref[...] = nsv[:, None]
    nd_ref[...] = ndv[:, None]


_norm_prescale = pl.pallas_call(
    _norm_prescale_body,
    grid=(NP // TB,),
    in_specs=[
        pl.BlockSpec((N_TILES, TB), lambda i: (0, i)),
        pl.BlockSpec((N_TILES, TB), lambda i: (0, i)),
        pl.BlockSpec((TB, D), lambda i: (i, 0)),
    ],
    out_specs=[
        pl.BlockSpec((TB, D), lambda i: (i, 0)),
        pl.BlockSpec((TB, 1), lambda i: (i, 0)),
        pl.BlockSpec((TB, 1), lambda i: (i, 0)),
    ],
    out_shape=[
        jax.ShapeDtypeStruct((NP, D), jnp.float32),
        jax.ShapeDtypeStruct((NP, 1), jnp.float32),
        jax.ShapeDtypeStruct((NP, 1), jnp.float32),
    ],
)


def _mid_body(acc_ref, nd_ref, ns_ref, w_ref, b_ref, t1_ref):
    a = (acc_ref[0] + acc_ref[1]) * nd_ref[...]
    h = jnp.dot(a, w_ref[...], preferred_element_type=jnp.float32) + b_ref[...]
    t1_ref[...] = h * ns_ref[...]


_mid_layer = pl.pallas_call(
    _mid_body,
    grid=(NP // TB,),
    in_specs=[
        pl.BlockSpec((2, TB, D), lambda i: (0, i, 0)),
        pl.BlockSpec((TB, 1), lambda i: (i, 0)),
        pl.BlockSpec((TB, 1), lambda i: (i, 0)),
        pl.BlockSpec((D, D), lambda i: (0, 0)),
        pl.BlockSpec((1, D), lambda i: (0, 0)),
    ],
    out_specs=pl.BlockSpec((TB, D), lambda i: (i, 0)),
    out_shape=jax.ShapeDtypeStruct((NP, D), jnp.float32),
)


def _out_body(acc_ref, nd_ref, w_ref, b_ref, o_ref):
    a = (acc_ref[0] + acc_ref[1]) * nd_ref[...]
    o_ref[...] = jnp.dot(a, w_ref[...], preferred_element_type=jnp.float32) + b_ref[...]


_out_layer = pl.pallas_call(
    _out_body,
    grid=(NP // TB,),
    in_specs=[
        pl.BlockSpec((2, TB, D), lambda i: (0, i, 0)),
        pl.BlockSpec((TB, 1), lambda i: (i, 0)),
        pl.BlockSpec((D, D), lambda i: (0, 0)),
        pl.BlockSpec((1, D), lambda i: (0, 0)),
    ],
    out_specs=pl.BlockSpec((TB, D), lambda i: (i, 0)),
    out_shape=jax.ShapeDtypeStruct((NP, D), jnp.float32),
)


def kernel(x, edge_index, W1, b1, W2, b2):
    src = edge_index[0].astype(jnp.int32)
    dst = edge_index[1].astype(jnp.int32)
    padi = N_NODES + jnp.arange(EP - N_EDGES, dtype=jnp.int32) % (NP - N_NODES)
    src3 = jnp.concatenate([src, padi]).reshape(N_TILES, NCH, CHUNK)
    dst3 = jnp.concatenate([dst, padi]).reshape(N_TILES, NCH, CHUNK)
    z1 = jnp.zeros((NP,), jnp.float32)
    z2 = jnp.zeros((RPS, D), jnp.float32)
    x_p = jnp.pad(x, ((0, NP - N_NODES), (0, 0)))

    hs, hd = _deg_kernel(src3, dst3, z1)
    t0, ns, nd = _norm_prescale(hs, hd, x_p)
    acc1 = _prop_kernel(t0, src3, dst3, z2)
    t1 = _mid_layer(acc1, nd, ns, W1, b1.reshape(1, D))
    acc2 = _prop_kernel(t1, src3, dst3, z2)
    out = _out_layer(acc2, nd, W2, b2.reshape(1, D))
    return out[:N_NODES]


# speedup vs baseline: 3.3087x; 1.0298x over previous
import functools

import jax
import jax.numpy as jnp
from jax import lax
from jax.experimental import pallas as pl
from jax.experimental.pallas import tpu as pltpu
from jax.experimental.pallas import tpu_sc as plsc

N_NODES = 10000
N_EDGES = 320000
D = 128

NP = 10240
N_TILES = 32
CHUNK = 128
NCHG = N_EDGES // CHUNK
NCH = 80
HCH = NCH // 2
RPS = NP // 16
NBUF = 2
TB = 512
TBO = 400

_mesh = plsc.VectorSubcoreMesh(core_axis_name="c", subcore_axis_name="s")
_sc_params = pltpu.CompilerParams(needs_layout_passes=False)


def _fill_safe(idx_v, nrows):
    lanes = lax.iota(jnp.int32, 16)

    def body(j, carry):
        base = j * CHUNK
        for u in range(CHUNK // 16):
            b0 = N_NODES + (base + u * 16) % 224
            idx_v[j, pl.ds(u * 16, 16)] = lanes + b0
        return carry

    lax.fori_loop(0, nrows, body, 0)


def _load_idx(ei_hbm, plane, w, half, idx_v):
    src = ei_hbm.at[plane]

    @pl.when(w < 30)
    def _():
        off = pl.multiple_of(w * NCH + half * HCH, 8)
        pltpu.sync_copy(src.at[pl.ds(off, HCH)], idx_v)

    if half == 0:

        @pl.when(w == 30)
        def _():
            pltpu.sync_copy(src.at[pl.ds(2400, HCH)], idx_v)

        @pl.when(w == 31)
        def _():
            pltpu.sync_copy(src.at[pl.ds(2472, 24)], idx_v.at[pl.ds(0, 24)])
            pltpu.sync_copy(src.at[pl.ds(2496, 4)], idx_v.at[pl.ds(24, 4)])
    else:

        @pl.when(w == 30)
        def _():
            pltpu.sync_copy(src.at[pl.ds(2440, 32)], idx_v.at[pl.ds(0, 32)])


@functools.partial(
    pl.kernel,
    out_type=(
        jax.ShapeDtypeStruct((N_TILES, NP), jnp.float32),
        jax.ShapeDtypeStruct((N_TILES, NP), jnp.float32),
    ),
    mesh=_mesh,
    scratch_types=[
        pltpu.VMEM((HCH, CHUNK), jnp.int32),
        pltpu.VMEM((HCH, CHUNK), jnp.int32),
        pltpu.VMEM((NP,), jnp.float32),
        pltpu.VMEM((NP,), jnp.float32),
    ],
    compiler_params=_sc_params,
)
def _deg_kernel(ei_hbm, z1_hbm, outs_hbm, outd_hbm, src_v, dst_v, hs_v, hd_v):
    w = lax.axis_index("s") * 2 + lax.axis_index("c")
    pltpu.sync_copy(z1_hbm, hs_v)
    pltpu.sync_copy(z1_hbm, hd_v)
    ones = jnp.ones((16,), jnp.float32)

    for half in range(2):

        @pl.when(w >= 30)
        def _():
            _fill_safe(src_v, HCH)
            _fill_safe(dst_v, HCH)

        _load_idx(ei_hbm, 0, w, half, src_v)
        _load_idx(ei_hbm, 1, w, half, dst_v)

        def body(j, carry):
            for u in range(CHUNK // 16):
                idx16 = src_v[j, pl.ds(u * 16, 16)]
                plsc.addupdate_scatter(hs_v, [idx16], ones)
                idx16 = dst_v[j, pl.ds(u * 16, 16)]
                plsc.addupdate_scatter(hd_v, [idx16], ones)
            return carry

        lax.fori_loop(0, HCH, body, 0)

    pltpu.sync_copy(hs_v, outs_hbm.at[w])
    pltpu.sync_copy(hd_v, outd_hbm.at[w])


@functools.partial(
    pl.kernel,
    out_type=jax.ShapeDtypeStruct((2, NP, D), jnp.float32),
    mesh=_mesh,
    scratch_types=[
        pltpu.VMEM((HCH, CHUNK), jnp.int32),
        pltpu.VMEM((HCH, CHUNK), jnp.int32),
        pltpu.VMEM((NBUF, CHUNK, D), jnp.float32),
        pltpu.VMEM_SHARED((NP, D), jnp.float32),
        pltpu.SemaphoreType.DMA,
        pltpu.SemaphoreType.DMA,
    ],
    compiler_params=_sc_params,
)
def _prop_kernel(t_hbm, ei_hbm, z2_hbm, out_hbm, si_v, di_v, rows_v, acc_sh,
                 sem0, sem1):
    sems = [sem0, sem1]
    c = lax.axis_index("c")
    s = lax.axis_index("s")
    w = s * 2 + c
    pltpu.sync_copy(z2_hbm, acc_sh.at[pl.ds(s * RPS, RPS)])

    def gstart(b, j):
        pltpu.async_copy(t_hbm.at[si_v.at[j]], rows_v.at[b], sems[b])

    def gwait(b):
        pltpu.make_async_copy(t_hbm.at[si_v.at[0]], rows_v.at[b],
                              sems[b]).wait()

    def step(t, b, with_issue):
        gwait(b)
        pltpu.sync_copy(rows_v.at[b], acc_sh.at[di_v.at[t]], add=True)
        if with_issue:
            gstart(b, t + 2)

    plsc.subcore_barrier()

    for half in range(2):

        @pl.when(w >= 30)
        def _():
            _fill_safe(si_v, HCH)
            _fill_safe(di_v, HCH)

        _load_idx(ei_hbm, 0, w, half, si_v)
        _load_idx(ei_hbm, 1, w, half, di_v)
        gstart(0, 0)
        gstart(1, 1)
        step(0, 0, True)

        def body(k, carry):
            for i in range(2):
                step(1 + k * 2 + i, (1 + i) % 2, True)
            return carry

        lax.fori_loop(0, (HCH - 4) // 2, body, 0)
        step(HCH - 3, (HCH - 3) % 2, True)
        step(HCH - 2, (HCH - 2) % 2, False)
        step(HCH - 1, (HCH - 1) % 2, False)

    plsc.subcore_barrier()
    pltpu.sync_copy(acc_sh.at[pl.ds(s * RPS, RPS)],
                    out_hbm.at[c, pl.ds(s * RPS, RPS)])


def _norm_prescale_body(ps_ref, pd_ref, x_ref, t0_ref, ns_ref, nd_ref):
    degs = jnp.sum(ps_ref[...], axis=0)
    degd = jnp.sum(pd_ref[...], axis=0)
    nsv = lax.rsqrt(jnp.maximum(degs, 1.0))
    ndv = lax.rsqrt(jnp.maximum(degd, 1.0))
    t0_ref[...] = x_ref[...] * nsv[:, None]
    ns_ref[...] = nsv[:, None]
    nd_ref[...] = ndv[:, None]


_norm_prescale = pl.pallas_call(
    _norm_prescale_body,
    grid=(NP // TB,),
    in_specs=[
        pl.BlockSpec((N_TILES, TB), lambda i: (0, i)),
        pl.BlockSpec((N_TILES, TB), lambda i: (0, i)),
        pl.BlockSpec((TB, D), lambda i: (i, 0)),
    ],
    out_specs=[
        pl.BlockSpec((TB, D), lambda i: (i, 0)),
        pl.BlockSpec((TB, 1), lambda i: (i, 0)),
        pl.BlockSpec((TB, 1), lambda i: (i, 0)),
    ],
    out_shape=[
        jax.ShapeDtypeStruct((NP, D), jnp.float32),
        jax.ShapeDtypeStruct((NP, 1), jnp.float32),
        jax.ShapeDtypeStruct((NP, 1), jnp.float32),
    ],
)


def _mid_body(acc_ref, nd_ref, ns_ref, w_ref, b_ref, t1_ref):
    a = (acc_ref[0] + acc_ref[1]) * nd_ref[...]
    h = jnp.dot(a, w_ref[...], preferred_element_type=jnp.float32) + b_ref[...]
    t1_ref[...] = h * ns_ref[...]


_mid_layer = pl.pallas_call(
    _mid_body,
    grid=(NP // TB,),
    in_specs=[
        pl.BlockSpec((2, TB, D), lambda i: (0, i, 0)),
        pl.BlockSpec((TB, 1), lambda i: (i, 0)),
        pl.BlockSpec((TB, 1), lambda i: (i, 0)),
        pl.BlockSpec((D, D), lambda i: (0, 0)),
        pl.BlockSpec((1, D), lambda i: (0, 0)),
    ],
    out_specs=pl.BlockSpec((TB, D), lambda i: (i, 0)),
    out_shape=jax.ShapeDtypeStruct((NP, D), jnp.float32),
)


def _out_body(acc_ref, nd_ref, w_ref, b_ref, o_ref):
    a = (acc_ref[0] + acc_ref[1]) * nd_ref[...]
    o_ref[...] = jnp.dot(a, w_ref[...], preferred_element_type=jnp.float32) + b_ref[...]


_out_layer = pl.pallas_call(
    _out_body,
    grid=(N_NODES // TBO,),
    in_specs=[
        pl.BlockSpec((2, TBO, D), lambda i: (0, i, 0)),
        pl.BlockSpec((TBO, 1), lambda i: (i, 0)),
        pl.BlockSpec((D, D), lambda i: (0, 0)),
        pl.BlockSpec((1, D), lambda i: (0, 0)),
    ],
    out_specs=pl.BlockSpec((TBO, D), lambda i: (i, 0)),
    out_shape=jax.ShapeDtypeStruct((N_NODES, D), jnp.float32),
)


def kernel(x, edge_index, W1, b1, W2, b2):
    ei3 = edge_index.astype(jnp.int32).reshape(2, NCHG, CHUNK)
    z1 = jnp.zeros((NP,), jnp.float32)
    z2 = jnp.zeros((RPS, D), jnp.float32)
    x_p = jnp.pad(x, ((0, NP - N_NODES), (0, 0)))

    hs, hd = _deg_kernel(ei3, z1)
    t0, ns, nd = _norm_prescale(hs, hd, x_p)
    acc1 = _prop_kernel(t0, ei3, z2)
    t1 = _mid_layer(acc1, nd, ns, W1, b1.reshape(1, D))
    acc2 = _prop_kernel(t1, ei3, z2)
    return _out_layer(acc2, nd, W2, b2.reshape(1, D))


# speedup vs baseline: 3.4083x; 1.0301x over previous
import functools

import jax
import jax.numpy as jnp
from jax import lax
from jax.experimental import pallas as pl
from jax.experimental.pallas import tpu as pltpu
from jax.experimental.pallas import tpu_sc as plsc

N_NODES = 10000
N_EDGES = 320000
D = 128

NP = 10240
N_TILES = 32
CHUNK = 128
NCHG = N_EDGES // CHUNK
NCH = 80
HCH = NCH // 2
RPS = NP // 16
NBUF = 2
TB = 1024
TBO = 400

_mesh = plsc.VectorSubcoreMesh(core_axis_name="c", subcore_axis_name="s")
_sc_params = pltpu.CompilerParams(needs_layout_passes=False)


def _fill_safe(idx_v, nrows):
    lanes = lax.iota(jnp.int32, 16)

    def body(j, carry):
        base = j * CHUNK
        for u in range(CHUNK // 16):
            b0 = N_NODES + (base + u * 16) % 224
            idx_v[j, pl.ds(u * 16, 16)] = lanes + b0
        return carry

    lax.fori_loop(0, nrows, body, 0)


def _load_idx(ei_hbm, plane, w, half, idx_v):
    src = ei_hbm.at[plane]

    @pl.when(w < 30)
    def _():
        off = pl.multiple_of(w * NCH + half * HCH, 8)
        pltpu.sync_copy(src.at[pl.ds(off, HCH)], idx_v)

    if half == 0:

        @pl.when(w == 30)
        def _():
            pltpu.sync_copy(src.at[pl.ds(2400, HCH)], idx_v)

        @pl.when(w == 31)
        def _():
            pltpu.sync_copy(src.at[pl.ds(2472, 24)], idx_v.at[pl.ds(0, 24)])
            pltpu.sync_copy(src.at[pl.ds(2496, 4)], idx_v.at[pl.ds(24, 4)])
    else:

        @pl.when(w == 30)
        def _():
            pltpu.sync_copy(src.at[pl.ds(2440, 32)], idx_v.at[pl.ds(0, 32)])


@functools.partial(
    pl.kernel,
    out_type=(
        jax.ShapeDtypeStruct((N_TILES, NP), jnp.float32),
        jax.ShapeDtypeStruct((N_TILES, NP), jnp.float32),
    ),
    mesh=_mesh,
    scratch_types=[
        pltpu.VMEM((HCH, CHUNK), jnp.int32),
        pltpu.VMEM((HCH, CHUNK), jnp.int32),
        pltpu.VMEM((NP,), jnp.float32),
        pltpu.VMEM((NP,), jnp.float32),
    ],
    compiler_params=_sc_params,
)
def _deg_kernel(ei_hbm, z1_hbm, outs_hbm, outd_hbm, src_v, dst_v, hs_v, hd_v):
    w = lax.axis_index("s") * 2 + lax.axis_index("c")
    pltpu.sync_copy(z1_hbm, hs_v)
    pltpu.sync_copy(z1_hbm, hd_v)
    ones = jnp.ones((16,), jnp.float32)

    for half in range(2):

        @pl.when(w >= 30)
        def _():
            _fill_safe(src_v, HCH)
            _fill_safe(dst_v, HCH)

        _load_idx(ei_hbm, 0, w, half, src_v)
        _load_idx(ei_hbm, 1, w, half, dst_v)

        def body(j, carry):
            for u in range(CHUNK // 16):
                idx16 = src_v[j, pl.ds(u * 16, 16)]
                plsc.addupdate_scatter(hs_v, [idx16], ones)
                idx16 = dst_v[j, pl.ds(u * 16, 16)]
                plsc.addupdate_scatter(hd_v, [idx16], ones)
            return carry

        lax.fori_loop(0, HCH, body, 0)

    pltpu.sync_copy(hs_v, outs_hbm.at[w])
    pltpu.sync_copy(hd_v, outd_hbm.at[w])


@functools.partial(
    pl.kernel,
    out_type=jax.ShapeDtypeStruct((2, NP, D), jnp.float32),
    mesh=_mesh,
    scratch_types=[
        pltpu.VMEM((HCH, CHUNK), jnp.int32),
        pltpu.VMEM((HCH, CHUNK), jnp.int32),
        pltpu.VMEM((NBUF, CHUNK, D), jnp.float32),
        pltpu.VMEM_SHARED((NP, D), jnp.float32),
        pltpu.SemaphoreType.DMA,
        pltpu.SemaphoreType.DMA,
    ],
    compiler_params=_sc_params,
)
def _prop_kernel(t_hbm, ei_hbm, z2_hbm, out_hbm, si_v, di_v, rows_v, acc_sh,
                 sem0, sem1):
    sems = [sem0, sem1]
    c = lax.axis_index("c")
    s = lax.axis_index("s")
    w = s * 2 + c
    pltpu.sync_copy(z2_hbm, acc_sh.at[pl.ds(s * RPS, RPS)])

    def gstart(b, j):
        pltpu.async_copy(t_hbm.at[si_v.at[j]], rows_v.at[b], sems[b])

    def gwait(b):
        pltpu.make_async_copy(t_hbm.at[si_v.at[0]], rows_v.at[b],
                              sems[b]).wait()

    def step(t, b, with_issue):
        gwait(b)
        pltpu.sync_copy(rows_v.at[b], acc_sh.at[di_v.at[t]], add=True)
        if with_issue:
            gstart(b, t + 2)

    plsc.subcore_barrier()

    for half in range(2):

        @pl.when(w >= 30)
        def _():
            _fill_safe(si_v, HCH)
            _fill_safe(di_v, HCH)

        _load_idx(ei_hbm, 0, w, half, si_v)
        _load_idx(ei_hbm, 1, w, half, di_v)
        gstart(0, 0)
        gstart(1, 1)
        step(0, 0, True)

        def body(k, carry):
            for i in range(2):
                step(1 + k * 2 + i, (1 + i) % 2, True)
            return carry

        lax.fori_loop(0, (HCH - 4) // 2, body, 0)
        step(HCH - 3, (HCH - 3) % 2, True)
        step(HCH - 2, (HCH - 2) % 2, False)
        step(HCH - 1, (HCH - 1) % 2, False)

    plsc.subcore_barrier()
    pltpu.sync_copy(acc_sh.at[pl.ds(s * RPS, RPS)],
                    out_hbm.at[c, pl.ds(s * RPS, RPS)])


def _norm_prescale_body(ps_ref, pd_ref, x_ref, t0_ref, ns_ref, nd_ref):
    degs = jnp.sum(ps_ref[...], axis=0)
    degd = jnp.sum(pd_ref[...], axis=0)
    nsv = lax.rsqrt(jnp.maximum(degs, 1.0))
    ndv = lax.rsqrt(jnp.maximum(degd, 1.0))
    t0_ref[...] = x_ref[...] * nsv[:, None]
    ns_ref[...] = nsv[:, None]
    nd_ref[...] = ndv[:, None]


_norm_prescale = pl.pallas_call(
    _norm_prescale_body,
    grid=(NP // TB,),
    in_specs=[
        pl.BlockSpec((N_TILES, TB), lambda i: (0, i)),
        pl.BlockSpec((N_TILES, TB), lambda i: (0, i)),
        pl.BlockSpec((TB, D), lambda i: (i, 0)),
    ],
    out_specs=[
        pl.BlockSpec((TB, D), lambda i: (i, 0)),
        pl.BlockSpec((TB, 1), lambda i: (i, 0)),
        pl.BlockSpec((TB, 1), lambda i: (i, 0)),
    ],
    out_shape=[
        jax.ShapeDtypeStruct((NP, D), jnp.float32),
        jax.ShapeDtypeStruct((NP, 1), jnp.float32),
        jax.ShapeDtypeStruct((NP, 1), jnp.float32),
    ],
)


def _mid_body(acc_ref, nd_ref, ns_ref, w_ref, b_ref, t1_ref):
    a = (acc_ref[0] + acc_ref[1]) * nd_ref[...]
    h = jnp.dot(a, w_ref[...], preferred_element_type=jnp.float32) + b_ref[...]
    t1_ref[...] = h * ns_ref[...]


_mid_layer = pl.pallas_call(
    _mid_body,
    grid=(NP // TB,),
    in_specs=[
        pl.BlockSpec((2, TB, D), lambda i: (0, i, 0)),
        pl.BlockSpec((TB, 1), lambda i: (i, 0)),
        pl.BlockSpec((TB, 1), lambda i: (i, 0)),
        pl.BlockSpec((D, D), lambda i: (0, 0)),
        pl.BlockSpec((1, D), lambda i: (0, 0)),
    ],
    out_specs=pl.BlockSpec((TB, D), lambda i: (i, 0)),
    out_shape=jax.ShapeDtypeStruct((NP, D), jnp.float32),
)


def _out_body(acc_ref, nd_ref, w_ref, b_ref, o_ref):
    a = (acc_ref[0] + acc_ref[1]) * nd_ref[...]
    o_ref[...] = jnp.dot(a, w_ref[...], preferred_element_type=jnp.float32) + b_ref[...]


_out_layer = pl.pallas_call(
    _out_body,
    grid=(N_NODES // TBO,),
    in_specs=[
        pl.BlockSpec((2, TBO, D), lambda i: (0, i, 0)),
        pl.BlockSpec((TBO, 1), lambda i: (i, 0)),
        pl.BlockSpec((D, D), lambda i: (0, 0)),
        pl.BlockSpec((1, D), lambda i: (0, 0)),
    ],
    out_specs=pl.BlockSpec((TBO, D), lambda i: (i, 0)),
    out_shape=jax.ShapeDtypeStruct((N_NODES, D), jnp.float32),
)


def kernel(x, edge_index, W1, b1, W2, b2):
    ei3 = edge_index.astype(jnp.int32).reshape(2, NCHG, CHUNK)
    z1 = jnp.zeros((NP,), jnp.float32)
    z2 = jnp.zeros((RPS, D), jnp.float32)
    x_p = jnp.pad(x, ((0, NP - N_NODES), (0, 0)))

    hs, hd = _deg_kernel(ei3, z1)
    t0, ns, nd = _norm_prescale(hs, hd, x_p)
    acc1 = _prop_kernel(t0, ei3, z2)
    t1 = _mid_layer(acc1, nd, ns, W1, b1.reshape(1, D))
    acc2 = _prop_kernel(t1, ei3, z2)
    return _out_layer(acc2, nd, W2, b2.reshape(1, D))


# speedup vs baseline: 3.6422x; 1.0686x over previous
import functools

import jax
import jax.numpy as jnp
from jax import lax
from jax.experimental import pallas as pl
from jax.experimental.pallas import tpu as pltpu
from jax.experimental.pallas import tpu_sc as plsc

N_NODES = 10000
N_EDGES = 320000
D = 128

NP = 10240
N_TILES = 32
CHUNK = 128
NCHG = N_EDGES // CHUNK
NCH = 80
HCH = NCH // 2
RPS = NP // 16
NBUF = 2
TB = 1024
TBO = 1000

_mesh = plsc.VectorSubcoreMesh(core_axis_name="c", subcore_axis_name="s")
_sc_params = pltpu.CompilerParams(needs_layout_passes=False)


def _fill_safe(idx_v, nrows):
    lanes = lax.iota(jnp.int32, 16)

    def body(j, carry):
        base = j * CHUNK
        for u in range(CHUNK // 16):
            b0 = N_NODES + (base + u * 16) % 224
            idx_v[j, pl.ds(u * 16, 16)] = lanes + b0
        return carry

    lax.fori_loop(0, nrows, body, 0)


def _load_idx(ei_hbm, plane, w, half, idx_v):
    src = ei_hbm.at[plane]

    @pl.when(w < 30)
    def _():
        off = pl.multiple_of(w * NCH + half * HCH, 8)
        pltpu.sync_copy(src.at[pl.ds(off, HCH)], idx_v)

    if half == 0:

        @pl.when(w == 30)
        def _():
            pltpu.sync_copy(src.at[pl.ds(2400, HCH)], idx_v)

        @pl.when(w == 31)
        def _():
            pltpu.sync_copy(src.at[pl.ds(2472, 24)], idx_v.at[pl.ds(0, 24)])
            pltpu.sync_copy(src.at[pl.ds(2496, 4)], idx_v.at[pl.ds(24, 4)])
    else:

        @pl.when(w == 30)
        def _():
            pltpu.sync_copy(src.at[pl.ds(2440, 32)], idx_v.at[pl.ds(0, 32)])


@functools.partial(
    pl.kernel,
    out_type=(
        jax.ShapeDtypeStruct((N_TILES, NP), jnp.float32),
        jax.ShapeDtypeStruct((N_TILES, NP), jnp.float32),
    ),
    mesh=_mesh,
    scratch_types=[
        pltpu.VMEM((HCH, CHUNK), jnp.int32),
        pltpu.VMEM((HCH, CHUNK), jnp.int32),
        pltpu.VMEM((NP,), jnp.float32),
        pltpu.VMEM((NP,), jnp.float32),
    ],
    compiler_params=_sc_params,
)
def _deg_kernel(ei_hbm, outs_hbm, outd_hbm, src_v, dst_v, hs_v, hd_v):
    w = lax.axis_index("s") * 2 + lax.axis_index("c")
    zero = jnp.zeros((16,), jnp.float32)

    def zbody(i, carry):
        hs_v[pl.ds(i * 16, 16)] = zero
        hd_v[pl.ds(i * 16, 16)] = zero
        return carry

    lax.fori_loop(0, NP // 16, zbody, 0)
    ones = jnp.ones((16,), jnp.float32)

    for half in range(2):

        @pl.when(w >= 30)
        def _():
            _fill_safe(src_v, HCH)
            _fill_safe(dst_v, HCH)

        _load_idx(ei_hbm, 0, w, half, src_v)
        _load_idx(ei_hbm, 1, w, half, dst_v)

        def body(j, carry):
            for u in range(CHUNK // 16):
                idx16 = src_v[j, pl.ds(u * 16, 16)]
                plsc.addupdate_scatter(hs_v, [idx16], ones)
                idx16 = dst_v[j, pl.ds(u * 16, 16)]
                plsc.addupdate_scatter(hd_v, [idx16], ones)
            return carry

        lax.fori_loop(0, HCH, body, 0)

    pltpu.sync_copy(hs_v, outs_hbm.at[w])
    pltpu.sync_copy(hd_v, outd_hbm.at[w])


@functools.partial(
    pl.kernel,
    out_type=jax.ShapeDtypeStruct((2, NP, D), jnp.float32),
    mesh=_mesh,
    scratch_types=[
        pltpu.VMEM((HCH, CHUNK), jnp.int32),
        pltpu.VMEM((HCH, CHUNK), jnp.int32),
        pltpu.VMEM((NBUF, CHUNK, D), jnp.float32),
        pltpu.VMEM_SHARED((NP, D), jnp.float32),
        pltpu.SemaphoreType.DMA,
        pltpu.SemaphoreType.DMA,
    ],
    compiler_params=_sc_params,
)
def _prop_kernel(t_hbm, ei_hbm, out_hbm, si_v, di_v, rows_v, acc_sh,
                 sem0, sem1):
    sems = [sem0, sem1]
    c = lax.axis_index("c")
    s = lax.axis_index("s")
    w = s * 2 + c
    zero = jnp.zeros((16,), jnp.float32)

    def zbody(i, carry):
        for u in range(D // 16):
            rows_v[0, i, pl.ds(u * 16, 16)] = zero
        return carry

    lax.fori_loop(0, CHUNK, zbody, 0)
    for m in range(RPS // CHUNK):
        pltpu.sync_copy(rows_v.at[0],
                        acc_sh.at[pl.ds(s * RPS + m * CHUNK, CHUNK)])

    def gstart(b, j):
        pltpu.async_copy(t_hbm.at[si_v.at[j]], rows_v.at[b], sems[b])

    def gwait(b):
        pltpu.make_async_copy(t_hbm.at[si_v.at[0]], rows_v.at[b],
                              sems[b]).wait()

    def step(t, b, with_issue):
        gwait(b)
        pltpu.sync_copy(rows_v.at[b], acc_sh.at[di_v.at[t]], add=True)
        if with_issue:
            gstart(b, t + 2)

    plsc.subcore_barrier()

    for half in range(2):

        @pl.when(w >= 30)
        def _():
            _fill_safe(si_v, HCH)
            _fill_safe(di_v, HCH)

        _load_idx(ei_hbm, 0, w, half, si_v)
        _load_idx(ei_hbm, 1, w, half, di_v)
        gstart(0, 0)
        gstart(1, 1)
        step(0, 0, True)

        def body(k, carry):
            for i in range(2):
                step(1 + k * 2 + i, (1 + i) % 2, True)
            return carry

        lax.fori_loop(0, (HCH - 4) // 2, body, 0)
        step(HCH - 3, (HCH - 3) % 2, True)
        step(HCH - 2, (HCH - 2) % 2, False)
        step(HCH - 1, (HCH - 1) % 2, False)

    plsc.subcore_barrier()
    pltpu.sync_copy(acc_sh.at[pl.ds(s * RPS, RPS)],
                    out_hbm.at[c, pl.ds(s * RPS, RPS)])


def _norm_prescale_body(ps_ref, pd_ref, x_ref, t0_ref, ns_ref, nd_ref):
    degs = jnp.sum(ps_ref[...], axis=0)
    degd = jnp.sum(pd_ref[...], axis=0)
    nsv = lax.rsqrt(jnp.maximum(degs, 1.0))
    ndv = lax.rsqrt(jnp.maximum(degd, 1.0))
    t0_ref[...] = x_ref[...] * nsv[:, None]
    ns_ref[...] = nsv[:, None]
    nd_ref[...] = ndv[:, None]


_norm_prescale = pl.pallas_call(
    _norm_prescale_body,
    grid=(NP // TB,),
    in_specs=[
        pl.BlockSpec((N_TILES, TB), lambda i: (0, i)),
        pl.BlockSpec((N_TILES, TB), lambda i: (0, i)),
        pl.BlockSpec((TB, D), lambda i: (i, 0)),
    ],
    out_specs=[
        pl.BlockSpec((TB, D), lambda i: (i, 0)),
        pl.BlockSpec((TB, 1), lambda i: (i, 0)),
        pl.BlockSpec((TB, 1), lambda i: (i, 0)),
    ],
    out_shape=[
        jax.ShapeDtypeStruct((NP, D), jnp.float32),
        jax.ShapeDtypeStruct((NP, 1), jnp.float32),
        jax.ShapeDtypeStruct((NP, 1), jnp.float32),
    ],
)


def _mid_body(acc_ref, nd_ref, ns_ref, w_ref, b_ref, t1_ref):
    a = (acc_ref[0] + acc_ref[1]) * nd_ref[...]
    h = jnp.dot(a, w_ref[...], preferred_element_type=jnp.float32) + b_ref[...]
    t1_ref[...] = h * ns_ref[...]


_mid_layer = pl.pallas_call(
    _mid_body,
    grid=(NP // TB,),
    in_specs=[
        pl.BlockSpec((2, TB, D), lambda i: (0, i, 0)),
        pl.BlockSpec((TB, 1), lambda i: (i, 0)),
        pl.BlockSpec((TB, 1), lambda i: (i, 0)),
        pl.BlockSpec((D, D), lambda i: (0, 0)),
        pl.BlockSpec((1, D), lambda i: (0, 0)),
    ],
    out_specs=pl.BlockSpec((TB, D), lambda i: (i, 0)),
    out_shape=jax.ShapeDtypeStruct((NP, D), jnp.float32),
)


def _out_body(acc_ref, nd_ref, w_ref, b_ref, o_ref):
    a = (acc_ref[0] + acc_ref[1]) * nd_ref[...]
    o_ref[...] = jnp.dot(a, w_ref[...], preferred_element_type=jnp.float32) + b_ref[...]


_out_layer = pl.pallas_call(
    _out_body,
    grid=(N_NODES // TBO,),
    in_specs=[
        pl.BlockSpec((2, TBO, D), lambda i: (0, i, 0)),
        pl.BlockSpec((TBO, 1), lambda i: (i, 0)),
        pl.BlockSpec((D, D), lambda i: (0, 0)),
        pl.BlockSpec((1, D), lambda i: (0, 0)),
    ],
    out_specs=pl.BlockSpec((TBO, D), lambda i: (i, 0)),
    out_shape=jax.ShapeDtypeStruct((N_NODES, D), jnp.float32),
)


def kernel(x, edge_index, W1, b1, W2, b2):
    ei3 = edge_index.astype(jnp.int32).reshape(2, NCHG, CHUNK)
    x_p = jnp.pad(x, ((0, NP - N_NODES), (0, 0)))

    hs, hd = _deg_kernel(ei3)
    t0, ns, nd = _norm_prescale(hs, hd, x_p)
    acc1 = _prop_kernel(t0, ei3)
    t1 = _mid_layer(acc1, nd, ns, W1, b1.reshape(1, D))
    acc2 = _prop_kernel(t1, ei3)
    return _out_layer(acc2, nd, W2, b2.reshape(1, D))
